# baseline (speedup 1.0000x reference)
import functools

import jax
import jax.numpy as jnp
from jax import lax
from jax.experimental import pallas as pl
from jax.experimental.pallas import tpu as pltpu
from jax.experimental.pallas import tpu_sc as plsc

_NC = 2
_NS = 16
_NW = _NC * _NS
_CH = 32
_GRP = 16


def _bn(y, g, b):
    m = jnp.mean(y, axis=0)
    v = jnp.mean((y - m) ** 2, axis=0)
    return (y - m) / jnp.sqrt(v + 1e-5) * g + b


def _tc_qkv(x, wq, wk, wv, scale):
    n, d = x.shape

    def body(x_ref, wq_ref, wk_ref, wv_ref, q_ref, kv_ref):
        xv = x_ref[...]
        q_ref[...] = jnp.dot(xv, wq_ref[...],
                             preferred_element_type=jnp.float32) * scale
        kv_ref[:, :d] = jnp.dot(xv, wk_ref[...],
                                preferred_element_type=jnp.float32)
        kv_ref[:, d:] = jnp.dot(xv, wv_ref[...],
                                preferred_element_type=jnp.float32)

    return pl.pallas_call(
        body,
        out_shape=(jax.ShapeDtypeStruct((n, d), jnp.float32),
                   jax.ShapeDtypeStruct((n, 2 * d), jnp.float32)),
    )(x, wq, wk, wv)


def _sc_edge_pass(q, kv, eidx, zeros, n, d, h, accw):
    ep = eidx.shape[1]
    epw = ep // _NW
    nchunk = epw // _CH
    ngrp = _CH // _GRP
    npad = ((n + 127) // 128) * 128
    npc = npad // _NS
    dh = d // h
    mesh = plsc.VectorSubcoreMesh(core_axis_name="c", subcore_axis_name="s")

    def set_types():
        return [
            pltpu.VMEM((_CH, d), jnp.float32),
            pltpu.VMEM((_CH, 2 * d), jnp.float32),
            pltpu.VMEM((3, _CH), jnp.int32),
            pltpu.VMEM((_CH,), jnp.int32),
            pltpu.VMEM((_CH, accw), jnp.float32),
            pltpu.SemaphoreType.DMA,
            pltpu.SemaphoreType.DMA,
            pltpu.SemaphoreType.DMA,
        ]

    @functools.partial(
        pl.kernel,
        out_type=jax.ShapeDtypeStruct((_NC, npad, accw), jnp.float32),
        mesh=mesh,
        compiler_params=pltpu.CompilerParams(
            use_tc_tiling_on_sc=False, needs_layout_passes=False),
        scratch_types=[
            pltpu.VMEM_SHARED((npad, accw), jnp.float32),
        ] + set_types() + set_types(),
    )
    def body(q_ref, kv_ref, eidx_ref, zero_ref, out_ref, acc, *bufs):
        sets = [bufs[:8], bufs[8:]]
        cid = lax.axis_index("c")
        sid = lax.axis_index("s")
        wid = cid * _NS + sid
        pltpu.sync_copy(zero_ref, acc.at[pl.ds(sid * npc, npc)])
        for S in sets:
            msg = S[4]
            for r in range(_CH):
                msg[r, pl.ds(d, 16)] = jnp.zeros((16,), jnp.float32)
        plsc.subcore_barrier()

        base0 = wid * epw
        iota = lax.iota(jnp.int32, 16)

        def fire_idx(i, S):
            (_, _, idxv, _, _, semi, _, _) = S
            b = base0 + i * _CH
            pltpu.async_copy(eidx_ref.at[:, pl.ds(b, _CH)], idxv, semi)

        def wait_idx(i, S):
            (_, _, idxv, _, _, semi, _, _) = S
            b = base0 + i * _CH
            pltpu.make_async_copy(eidx_ref.at[:, pl.ds(b, _CH)], idxv,
                                  semi).wait()

        def fire_gathers(S):
            (qr, kvr, idxv, _, _, _, semg, _) = S
            pltpu.async_copy(q_ref.at[idxv.at[1]], qr, semg)
            pltpu.async_copy(kv_ref.at[idxv.at[0]], kvr, semg)

        def wait_gathers(S):
            (qr, kvr, idxv, _, _, _, semg, _) = S
            pltpu.make_async_copy(q_ref.at[idxv.at[1]], qr, semg).wait()
            pltpu.make_async_copy(kv_ref.at[idxv.at[0]], kvr, semg).wait()

        def wait_scatter(S):
            (_, _, _, hold, msg, _, _, sems) = S
            pltpu.make_async_copy(msg, acc.at[hold], sems).wait()

        def compute_and_scatter(S):
            (qr, kvr, idxv, hold, msg, _, _, sems) = S
            for cc in range(_CH // 16):
                hold[pl.ds(cc * 16, 16)] = idxv[2, pl.ds(cc * 16, 16)]

            @pl.loop(0, ngrp)
            def _grp(g):
                rows = g * _GRP + iota
                for hh in range(h):
                    s = jnp.zeros((16,), jnp.float32)
                    for j in range(dh):
                        col = jnp.full((16,), hh * dh + j, jnp.int32)
                        a = plsc.load_gather(qr, [rows, col])
                        b = plsc.load_gather(kvr, [rows, col])
                        s = s + a * b
                    eh = jnp.exp(s)
                    plsc.store_scatter(msg, [rows, jnp.full((16,), d + hh,
                                                            jnp.int32)], eh)
                    for j in range(dh):
                        col = jnp.full((16,), hh * dh + j, jnp.int32)
                        vv = plsc.load_gather(kvr, [rows, d + col])
                        plsc.store_scatter(msg, [rows, col], eh * vv)
            pltpu.async_copy(msg, acc.at[hold], sems, add=True)

        fire_idx(0, sets[0])
        fire_idx(1, sets[1])
        wait_idx(0, sets[0])
        fire_gathers(sets[0])

        @pl.loop(0, nchunk // 2)
        def _pair(it):
            for ph in range(2):
                S, Sn = sets[ph], sets[1 - ph]
                i = it * 2 + ph
                wait_gathers(S)

                @pl.when(i >= 2)
                def _():
                    wait_scatter(S)

                @pl.when(i + 1 < nchunk)
                def _():
                    wait_idx(i + 1, Sn)
                    fire_gathers(Sn)

                @pl.when(i + 2 < nchunk)
                def _():
                    fire_idx(i + 2, S)

                compute_and_scatter(S)

        wait_scatter(sets[0])
        wait_scatter(sets[1])
        plsc.subcore_barrier()
        pltpu.sync_copy(acc.at[pl.ds(sid * npc, npc)],
                        out_ref.at[cid, pl.ds(sid * npc, npc)])

    return body(q, kv, eidx, zeros)


def _tc_dense(x, acc2, wo, ga, ba, g1, be1, w1, bb1, w2, bb2, g2, be2):
    n, d = x.shape
    h = 8
    dh = d // h

    def body(x_ref, acc_ref, wo_ref, ga_ref, ba_ref, g1_ref, be1_ref,
             w1_ref, bb1_ref, w2_ref, bb2_ref, g2_ref, be2_ref, out_ref):
        xv = x_ref[...]
        u = acc_ref[0, :n, :d] + acc_ref[1, :n, :d]
        den = acc_ref[0, :n, d:d + h] + acc_ref[1, :n, d:d + h]
        ind = (lax.broadcasted_iota(jnp.int32, (h, d), 1) // dh
               == lax.broadcasted_iota(jnp.int32, (h, d), 0)
               ).astype(jnp.float32)
        denf = jnp.dot(den, ind, preferred_element_type=jnp.float32)
        agg = u / (denf + 1e-16)
        y0 = jnp.dot(agg, wo_ref[...], preferred_element_type=jnp.float32) + xv
        a1 = _bn(y0, ga_ref[...], ba_ref[...])
        x1 = _bn(a1 + xv, g1_ref[...], be1_ref[...])
        hid = jnp.maximum(
            jnp.dot(x1, w1_ref[...], preferred_element_type=jnp.float32)
            + bb1_ref[...], 0.0)
        h2 = jnp.dot(hid, w2_ref[...],
                     preferred_element_type=jnp.float32) + bb2_ref[...]
        out_ref[...] = _bn(h2 + x1, g2_ref[...], be2_ref[...])

    return pl.pallas_call(
        body,
        out_shape=jax.ShapeDtypeStruct((n, d), jnp.float32),
    )(x, acc2, wo, ga, ba, g1, be1, w1, bb1, w2, bb2, g2, be2)


def _tc_final(x, wp, bp):
    n, d = x.shape
    c = wp.shape[1]

    def body(x_ref, wp_ref, bp_ref, out_ref):
        out_ref[...] = (jnp.dot(x_ref[...], wp_ref[...],
                                preferred_element_type=jnp.float32)
                        + bp_ref[...])

    return pl.pallas_call(
        body,
        out_shape=jax.ShapeDtypeStruct((n, c), jnp.float32),
    )(x, wp, bp)


def kernel(x, edge_index, Wq, Wk, Wv, Wo, gamma_attn, beta_attn, gamma1,
           beta1, W1, b1, W2, b2, gamma2, beta2, Wp, bp):
    n, d = x.shape
    nl = Wq.shape[0]
    h = 8
    dh = d // h
    accw = d + 16
    scale = 1.0 / (dh ** 0.5)
    src = edge_index[0]
    dst = edge_index[1]
    e = src.shape[0]
    ep = -(-e // (_NW * 2 * _CH)) * (_NW * 2 * _CH)
    pad = ep - e
    srcp = jnp.concatenate([src, jnp.zeros((pad,), jnp.int32)])
    dstg = jnp.concatenate([dst, jnp.zeros((pad,), jnp.int32)])
    dsts = jnp.concatenate([dst, jnp.full((pad,), n, jnp.int32)])
    eidx = jnp.stack([srcp, dstg, dsts])
    npad = ((n + 127) // 128) * 128
    zeros = jnp.zeros((npad // _NS, accw), jnp.float32)
    for l in range(nl):
        q, kv = _tc_qkv(x, Wq[l], Wk[l], Wv[l], scale)
        acc2 = _sc_edge_pass(q, kv, eidx, zeros, n, d, h, accw)
        x = _tc_dense(x, acc2, Wo[l], gamma_attn[l], beta_attn[l],
                      gamma1[l], beta1[l], W1[l], b1[l], W2[l], b2[l],
                      gamma2[l], beta2[l])
    return _tc_final(x, Wp, bp)


# speedup vs baseline: 1.0059x; 1.0059x over previous
import functools

import jax
import jax.numpy as jnp
from jax import lax
from jax.experimental import pallas as pl
from jax.experimental.pallas import tpu as pltpu
from jax.experimental.pallas import tpu_sc as plsc

_NC = 2
_NS = 16
_NW = _NC * _NS
_CH = 32
_GRP = 16


def _bn(y, g, b):
    m = jnp.mean(y, axis=0)
    v = jnp.mean((y - m) ** 2, axis=0)
    return (y - m) / jnp.sqrt(v + 1e-5) * g + b


def _tc_qkv(x, wq, wk, wv, scale):
    n, d = x.shape

    def body(x_ref, wq_ref, wk_ref, wv_ref, q_ref, kv_ref):
        xv = x_ref[...]
        q_ref[...] = jnp.dot(xv, wq_ref[...],
                             preferred_element_type=jnp.float32) * scale
        kv_ref[:, :d] = jnp.dot(xv, wk_ref[...],
                                preferred_element_type=jnp.float32)
        kv_ref[:, d:] = jnp.dot(xv, wv_ref[...],
                                preferred_element_type=jnp.float32)

    return pl.pallas_call(
        body,
        out_shape=(jax.ShapeDtypeStruct((n, d), jnp.float32),
                   jax.ShapeDtypeStruct((n, 2 * d), jnp.float32)),
    )(x, wq, wk, wv)


def _sc_edge_pass(q, kv, eidx, zeros, n, d, h, accw):
    ep = eidx.shape[1]
    epw = ep // _NW
    nchunk = epw // _CH
    ngrp = _CH // _GRP
    npad = ((n + 127) // 128) * 128
    npc = npad // _NS
    dh = d // h
    mesh = plsc.VectorSubcoreMesh(core_axis_name="c", subcore_axis_name="s")

    def set_types():
        return [
            pltpu.VMEM((_CH, d), jnp.float32),
            pltpu.VMEM((_CH, 2 * d), jnp.float32),
            pltpu.VMEM((3, _CH), jnp.int32),
            pltpu.VMEM((_CH,), jnp.int32),
            pltpu.VMEM((_CH, accw), jnp.float32),
            pltpu.SemaphoreType.DMA,
            pltpu.SemaphoreType.DMA,
            pltpu.SemaphoreType.DMA,
        ]

    @functools.partial(
        pl.kernel,
        out_type=jax.ShapeDtypeStruct((_NC, npad, accw), jnp.float32),
        mesh=mesh,
        compiler_params=pltpu.CompilerParams(
            use_tc_tiling_on_sc=False, needs_layout_passes=False),
        scratch_types=[
            pltpu.VMEM_SHARED((npad, accw), jnp.float32),
        ] + set_types() + set_types(),
    )
    def body(q_ref, kv_ref, eidx_ref, zero_ref, out_ref, acc, *bufs):
        sets = [bufs[:8], bufs[8:]]
        cid = lax.axis_index("c")
        sid = lax.axis_index("s")
        wid = cid * _NS + sid
        pltpu.sync_copy(zero_ref, acc.at[pl.ds(sid * npc, npc)])
        for S in sets:
            msg = S[4]
            for r in range(_CH):
                msg[r, pl.ds(d, 16)] = jnp.zeros((16,), jnp.float32)
        plsc.subcore_barrier()

        base0 = wid * epw
        iota = lax.iota(jnp.int32, 16)

        def fire_idx(i, S):
            (_, _, idxv, _, _, semi, _, _) = S
            b = base0 + i * _CH
            pltpu.async_copy(eidx_ref.at[:, pl.ds(b, _CH)], idxv, semi)

        def wait_idx(i, S):
            (_, _, idxv, _, _, semi, _, _) = S
            b = base0 + i * _CH
            pltpu.make_async_copy(eidx_ref.at[:, pl.ds(b, _CH)], idxv,
                                  semi).wait()

        def fire_gathers(S):
            (qr, kvr, idxv, _, _, _, semg, _) = S
            pltpu.async_copy(q_ref.at[idxv.at[1]], qr, semg)
            pltpu.async_copy(kv_ref.at[idxv.at[0]], kvr, semg)

        def wait_gathers(S):
            (qr, kvr, idxv, _, _, _, semg, _) = S
            pltpu.make_async_copy(q_ref.at[idxv.at[1]], qr, semg).wait()
            pltpu.make_async_copy(kv_ref.at[idxv.at[0]], kvr, semg).wait()

        def wait_scatter(S):
            (_, _, _, hold, msg, _, _, sems) = S
            pltpu.make_async_copy(msg, acc.at[hold], sems).wait()

        def compute_and_scatter(S):
            (qr, kvr, idxv, hold, msg, _, _, sems) = S
            for cc in range(_CH // 16):
                hold[pl.ds(cc * 16, 16)] = idxv[2, pl.ds(cc * 16, 16)]

            @pl.loop(0, ngrp)
            def _grp(g):
                rows = g * _GRP + iota
                for hh in range(h):
                    s0 = jnp.zeros((16,), jnp.float32)
                    s1 = jnp.zeros((16,), jnp.float32)
                    for j in range(0, dh, 2):
                        col = jnp.full((16,), hh * dh + j, jnp.int32)
                        col1 = jnp.full((16,), hh * dh + j + 1, jnp.int32)
                        a0 = plsc.load_gather(qr, [rows, col])
                        b0 = plsc.load_gather(kvr, [rows, col])
                        a1 = plsc.load_gather(qr, [rows, col1])
                        b1 = plsc.load_gather(kvr, [rows, col1])
                        s0 = s0 + a0 * b0
                        s1 = s1 + a1 * b1
                    eh = jnp.exp(s0 + s1)
                    plsc.store_scatter(msg, [rows, jnp.full((16,), d + hh,
                                                            jnp.int32)], eh)
                    for j in range(dh):
                        col = jnp.full((16,), hh * dh + j, jnp.int32)
                        vv = plsc.load_gather(kvr, [rows, d + col])
                        plsc.store_scatter(msg, [rows, col], eh * vv)
            pltpu.async_copy(msg, acc.at[hold], sems, add=True)

        fire_idx(0, sets[0])
        fire_idx(1, sets[1])
        wait_idx(0, sets[0])
        fire_gathers(sets[0])

        @pl.loop(0, nchunk // 2)
        def _pair(it):
            for ph in range(2):
                S, Sn = sets[ph], sets[1 - ph]
                i = it * 2 + ph
                wait_gathers(S)

                @pl.when(i >= 2)
                def _():
                    wait_scatter(S)

                @pl.when(i + 1 < nchunk)
                def _():
                    wait_idx(i + 1, Sn)
                    fire_gathers(Sn)

                @pl.when(i + 2 < nchunk)
                def _():
                    fire_idx(i + 2, S)

                compute_and_scatter(S)

        wait_scatter(sets[0])
        wait_scatter(sets[1])
        plsc.subcore_barrier()
        pltpu.sync_copy(acc.at[pl.ds(sid * npc, npc)],
                        out_ref.at[cid, pl.ds(sid * npc, npc)])

    return body(q, kv, eidx, zeros)


def _tc_dense(x, acc2, wo, ga, ba, g1, be1, w1, bb1, w2, bb2, g2, be2):
    n, d = x.shape
    h = 8
    dh = d // h

    def body(x_ref, acc_ref, wo_ref, ga_ref, ba_ref, g1_ref, be1_ref,
             w1_ref, bb1_ref, w2_ref, bb2_ref, g2_ref, be2_ref, out_ref):
        xv = x_ref[...]
        u = acc_ref[0, :n, :d] + acc_ref[1, :n, :d]
        den = acc_ref[0, :n, d:d + h] + acc_ref[1, :n, d:d + h]
        ind = (lax.broadcasted_iota(jnp.int32, (h, d), 1) // dh
               == lax.broadcasted_iota(jnp.int32, (h, d), 0)
               ).astype(jnp.float32)
        denf = jnp.dot(den, ind, preferred_element_type=jnp.float32)
        agg = u / (denf + 1e-16)
        y0 = jnp.dot(agg, wo_ref[...], preferred_element_type=jnp.float32) + xv
        a1 = _bn(y0, ga_ref[...], ba_ref[...])
        x1 = _bn(a1 + xv, g1_ref[...], be1_ref[...])
        hid = jnp.maximum(
            jnp.dot(x1, w1_ref[...], preferred_element_type=jnp.float32)
            + bb1_ref[...], 0.0)
        h2 = jnp.dot(hid, w2_ref[...],
                     preferred_element_type=jnp.float32) + bb2_ref[...]
        out_ref[...] = _bn(h2 + x1, g2_ref[...], be2_ref[...])

    return pl.pallas_call(
        body,
        out_shape=jax.ShapeDtypeStruct((n, d), jnp.float32),
    )(x, acc2, wo, ga, ba, g1, be1, w1, bb1, w2, bb2, g2, be2)


def _tc_final(x, wp, bp):
    n, d = x.shape
    c = wp.shape[1]

    def body(x_ref, wp_ref, bp_ref, out_ref):
        out_ref[...] = (jnp.dot(x_ref[...], wp_ref[...],
                                preferred_element_type=jnp.float32)
                        + bp_ref[...])

    return pl.pallas_call(
        body,
        out_shape=jax.ShapeDtypeStruct((n, c), jnp.float32),
    )(x, wp, bp)


def kernel(x, edge_index, Wq, Wk, Wv, Wo, gamma_attn, beta_attn, gamma1,
           beta1, W1, b1, W2, b2, gamma2, beta2, Wp, bp):
    n, d = x.shape
    nl = Wq.shape[0]
    h = 8
    dh = d // h
    accw = d + 16
    scale = 1.0 / (dh ** 0.5)
    src = edge_index[0]
    dst = edge_index[1]
    e = src.shape[0]
    ep = -(-e // (_NW * 2 * _CH)) * (_NW * 2 * _CH)
    pad = ep - e
    srcp = jnp.concatenate([src, jnp.zeros((pad,), jnp.int32)])
    dstg = jnp.concatenate([dst, jnp.zeros((pad,), jnp.int32)])
    dsts = jnp.concatenate([dst, jnp.full((pad,), n, jnp.int32)])
    eidx = jnp.stack([srcp, dstg, dsts])
    npad = ((n + 127) // 128) * 128
    zeros = jnp.zeros((npad // _NS, accw), jnp.float32)
    for l in range(nl):
        q, kv = _tc_qkv(x, Wq[l], Wk[l], Wv[l], scale)
        acc2 = _sc_edge_pass(q, kv, eidx, zeros, n, d, h, accw)
        x = _tc_dense(x, acc2, Wo[l], gamma_attn[l], beta_attn[l],
                      gamma1[l], beta1[l], W1[l], b1[l], W2[l], b2[l],
                      gamma2[l], beta2[l])
    return _tc_final(x, Wp, bp)


# speedup vs baseline: 1.7845x; 1.7741x over previous
import functools

import jax
import jax.numpy as jnp
from jax import lax
from jax.experimental import pallas as pl
from jax.experimental.pallas import tpu as pltpu
from jax.experimental.pallas import tpu_sc as plsc

_NC = 2
_NS = 16
_NW = _NC * _NS
_CH = 32
_GRP = 16


def _bn(y, g, b):
    m = jnp.mean(y, axis=0)
    v = jnp.mean((y - m) ** 2, axis=0)
    return (y - m) / jnp.sqrt(v + 1e-5) * g + b


def _tc_qkv(x, wq, wk, wv, scale):
    n, d = x.shape

    def body(x_ref, wq_ref, wk_ref, wv_ref, q_ref, kv_ref):
        xv = x_ref[...]
        q_ref[...] = jnp.dot(xv, wq_ref[...],
                             preferred_element_type=jnp.float32) * scale
        kv_ref[:, :d] = jnp.dot(xv, wk_ref[...],
                                preferred_element_type=jnp.float32)
        kv_ref[:, d:] = jnp.dot(xv, wv_ref[...],
                                preferred_element_type=jnp.float32)

    return pl.pallas_call(
        body,
        out_shape=(jax.ShapeDtypeStruct((n, d), jnp.float32),
                   jax.ShapeDtypeStruct((n, 2 * d), jnp.float32)),
    )(x, wq, wk, wv)


def _sc_edge_pass(q, kv, eidx, zeros, n, d, h, accw):
    ep = eidx.shape[1]
    epw = ep // _NW
    nchunk = epw // _CH
    ngrp = _CH // _GRP
    npad = ((n + 127) // 128) * 128
    npc = npad // _NS
    dh = d // h
    mesh = plsc.VectorSubcoreMesh(core_axis_name="c", subcore_axis_name="s")

    def set_types():
        return [
            pltpu.VMEM((_CH, d), jnp.float32),
            pltpu.VMEM((_CH, 2 * d), jnp.float32),
            pltpu.VMEM((3, _CH), jnp.int32),
            pltpu.VMEM((_CH,), jnp.int32),
            pltpu.VMEM((_CH, accw), jnp.float32),
            pltpu.SemaphoreType.DMA,
            pltpu.SemaphoreType.DMA,
            pltpu.SemaphoreType.DMA,
        ]

    @functools.partial(
        pl.kernel,
        out_type=jax.ShapeDtypeStruct((_NC, npad, accw), jnp.float32),
        mesh=mesh,
        compiler_params=pltpu.CompilerParams(
            use_tc_tiling_on_sc=False, needs_layout_passes=False),
        scratch_types=[
            pltpu.VMEM_SHARED((npad, accw), jnp.float32),
        ] + set_types() + set_types(),
    )
    def body(q_ref, kv_ref, eidx_ref, zero_ref, out_ref, acc, *bufs):
        sets = [bufs[:8], bufs[8:]]
        cid = lax.axis_index("c")
        sid = lax.axis_index("s")
        wid = cid * _NS + sid
        pltpu.sync_copy(zero_ref, acc.at[pl.ds(sid * npc, npc)])
        for S in sets:
            msg = S[4]
            for r in range(_CH):
                msg[r, pl.ds(d, 16)] = jnp.zeros((16,), jnp.float32)
        plsc.subcore_barrier()

        base0 = wid * epw
        iota = lax.iota(jnp.int32, 16)

        def fire_idx(i, S):
            (_, _, idxv, _, _, semi, _, _) = S
            b = base0 + i * _CH
            pltpu.async_copy(eidx_ref.at[:, pl.ds(b, _CH)], idxv, semi)

        def wait_idx(i, S):
            (_, _, idxv, _, _, semi, _, _) = S
            b = base0 + i * _CH
            pltpu.make_async_copy(eidx_ref.at[:, pl.ds(b, _CH)], idxv,
                                  semi).wait()

        def fire_gathers(S):
            (qr, kvr, idxv, _, _, _, semg, _) = S
            pltpu.async_copy(q_ref.at[idxv.at[1]], qr, semg)
            pltpu.async_copy(kv_ref.at[idxv.at[0]], kvr, semg)

        def wait_gathers(S):
            (qr, kvr, idxv, _, _, _, semg, _) = S
            pltpu.make_async_copy(q_ref.at[idxv.at[1]], qr, semg).wait()
            pltpu.make_async_copy(kv_ref.at[idxv.at[0]], kvr, semg).wait()

        def wait_scatter(S):
            (_, _, _, hold, msg, _, _, sems) = S
            pltpu.make_async_copy(msg, acc.at[hold], sems).wait()

        def compute_and_scatter(S):
            (qr, kvr, idxv, hold, msg, _, _, sems) = S
            for cc in range(_CH // 16):
                hold[pl.ds(cc * 16, 16)] = idxv[2, pl.ds(cc * 16, 16)]

            @pl.loop(0, ngrp)
            def _grp(g):
                rows = g * _GRP + iota
                for hh in range(h):
                    s = jnp.zeros((16,), jnp.float32)
                    for j in range(dh):
                        col = hh * dh + ((iota + j) & (dh - 1))
                        a = plsc.load_gather(qr, [rows, col])
                        b = plsc.load_gather(kvr, [rows, col])
                        s = s + a * b
                    eh = jnp.exp(s)
                    plsc.store_scatter(msg, [rows, jnp.full((16,), d + hh,
                                                            jnp.int32)], eh)
                    for j in range(dh):
                        col = jnp.full((16,), hh * dh + j, jnp.int32)
                        vv = plsc.load_gather(kvr, [rows, d + col])
                        plsc.store_scatter(msg, [rows, col], eh * vv)
            pltpu.async_copy(msg, acc.at[hold], sems, add=True)

        fire_idx(0, sets[0])
        fire_idx(1, sets[1])
        wait_idx(0, sets[0])
        fire_gathers(sets[0])

        @pl.loop(0, nchunk // 2)
        def _pair(it):
            for ph in range(2):
                S, Sn = sets[ph], sets[1 - ph]
                i = it * 2 + ph
                wait_gathers(S)

                @pl.when(i >= 2)
                def _():
                    wait_scatter(S)

                @pl.when(i + 1 < nchunk)
                def _():
                    wait_idx(i + 1, Sn)
                    fire_gathers(Sn)

                @pl.when(i + 2 < nchunk)
                def _():
                    fire_idx(i + 2, S)

                compute_and_scatter(S)

        wait_scatter(sets[0])
        wait_scatter(sets[1])
        plsc.subcore_barrier()
        pltpu.sync_copy(acc.at[pl.ds(sid * npc, npc)],
                        out_ref.at[cid, pl.ds(sid * npc, npc)])

    return body(q, kv, eidx, zeros)


def _tc_dense(x, acc2, wo, ga, ba, g1, be1, w1, bb1, w2, bb2, g2, be2):
    n, d = x.shape
    h = 8
    dh = d // h

    def body(x_ref, acc_ref, wo_ref, ga_ref, ba_ref, g1_ref, be1_ref,
             w1_ref, bb1_ref, w2_ref, bb2_ref, g2_ref, be2_ref, out_ref):
        xv = x_ref[...]
        u = acc_ref[0, :n, :d] + acc_ref[1, :n, :d]
        den = acc_ref[0, :n, d:d + h] + acc_ref[1, :n, d:d + h]
        ind = (lax.broadcasted_iota(jnp.int32, (h, d), 1) // dh
               == lax.broadcasted_iota(jnp.int32, (h, d), 0)
               ).astype(jnp.float32)
        denf = jnp.dot(den, ind, preferred_element_type=jnp.float32)
        agg = u / (denf + 1e-16)
        y0 = jnp.dot(agg, wo_ref[...], preferred_element_type=jnp.float32) + xv
        a1 = _bn(y0, ga_ref[...], ba_ref[...])
        x1 = _bn(a1 + xv, g1_ref[...], be1_ref[...])
        hid = jnp.maximum(
            jnp.dot(x1, w1_ref[...], preferred_element_type=jnp.float32)
            + bb1_ref[...], 0.0)
        h2 = jnp.dot(hid, w2_ref[...],
                     preferred_element_type=jnp.float32) + bb2_ref[...]
        out_ref[...] = _bn(h2 + x1, g2_ref[...], be2_ref[...])

    return pl.pallas_call(
        body,
        out_shape=jax.ShapeDtypeStruct((n, d), jnp.float32),
    )(x, acc2, wo, ga, ba, g1, be1, w1, bb1, w2, bb2, g2, be2)


def _tc_final(x, wp, bp):
    n, d = x.shape
    c = wp.shape[1]

    def body(x_ref, wp_ref, bp_ref, out_ref):
        out_ref[...] = (jnp.dot(x_ref[...], wp_ref[...],
                                preferred_element_type=jnp.float32)
                        + bp_ref[...])

    return pl.pallas_call(
        body,
        out_shape=jax.ShapeDtypeStruct((n, c), jnp.float32),
    )(x, wp, bp)


def kernel(x, edge_index, Wq, Wk, Wv, Wo, gamma_attn, beta_attn, gamma1,
           beta1, W1, b1, W2, b2, gamma2, beta2, Wp, bp):
    n, d = x.shape
    nl = Wq.shape[0]
    h = 8
    dh = d // h
    accw = d + 16
    scale = 1.0 / (dh ** 0.5)
    src = edge_index[0]
    dst = edge_index[1]
    e = src.shape[0]
    ep = -(-e // (_NW * 2 * _CH)) * (_NW * 2 * _CH)
    pad = ep - e
    srcp = jnp.concatenate([src, jnp.zeros((pad,), jnp.int32)])
    dstg = jnp.concatenate([dst, jnp.zeros((pad,), jnp.int32)])
    dsts = jnp.concatenate([dst, jnp.full((pad,), n, jnp.int32)])
    eidx = jnp.stack([srcp, dstg, dsts])
    npad = ((n + 127) // 128) * 128
    zeros = jnp.zeros((npad // _NS, accw), jnp.float32)
    for l in range(nl):
        q, kv = _tc_qkv(x, Wq[l], Wk[l], Wv[l], scale)
        acc2 = _sc_edge_pass(q, kv, eidx, zeros, n, d, h, accw)
        x = _tc_dense(x, acc2, Wo[l], gamma_attn[l], beta_attn[l],
                      gamma1[l], beta1[l], W1[l], b1[l], W2[l], b2[l],
                      gamma2[l], beta2[l])
    return _tc_final(x, Wp, bp)


# speedup vs baseline: 2.2340x; 1.2518x over previous
import functools

import jax
import jax.numpy as jnp
from jax import lax
from jax.experimental import pallas as pl
from jax.experimental.pallas import tpu as pltpu
from jax.experimental.pallas import tpu_sc as plsc

_NC = 2
_NS = 16
_NW = _NC * _NS
_CH = 32
_GRP = 16


def _bn(y, g, b):
    m = jnp.mean(y, axis=0)
    v = jnp.mean((y - m) ** 2, axis=0)
    return (y - m) / jnp.sqrt(v + 1e-5) * g + b


def _tc_qkv(x, wq, wk, wv, scale):
    n, d = x.shape

    def body(x_ref, wq_ref, wk_ref, wv_ref, q_ref, kv_ref):
        xv = x_ref[...]
        q_ref[...] = jnp.dot(xv, wq_ref[...],
                             preferred_element_type=jnp.float32) * scale
        kv_ref[:, :d] = jnp.dot(xv, wk_ref[...],
                                preferred_element_type=jnp.float32)
        kv_ref[:, d:] = jnp.dot(xv, wv_ref[...],
                                preferred_element_type=jnp.float32)

    return pl.pallas_call(
        body,
        out_shape=(jax.ShapeDtypeStruct((n, d), jnp.float32),
                   jax.ShapeDtypeStruct((n, 2 * d), jnp.float32)),
    )(x, wq, wk, wv)


def _sc_edge_pass(q, kv, eidx, zeros, n, d, h, accw):
    ep = eidx.shape[1]
    epw = ep // _NW
    nchunk = epw // _CH
    ngrp = _CH // _GRP
    npad = ((n + 127) // 128) * 128
    npc = npad // _NS
    dh = d // h
    mesh = plsc.VectorSubcoreMesh(core_axis_name="c", subcore_axis_name="s")

    def set_types():
        return [
            pltpu.VMEM((_CH, d), jnp.float32),
            pltpu.VMEM((_CH, 2 * d), jnp.float32),
            pltpu.VMEM((3, _CH), jnp.int32),
            pltpu.SemaphoreType.DMA,
            pltpu.SemaphoreType.DMA,
        ]

    @functools.partial(
        pl.kernel,
        out_type=jax.ShapeDtypeStruct((_NC, npad, accw), jnp.float32),
        mesh=mesh,
        compiler_params=pltpu.CompilerParams(
            use_tc_tiling_on_sc=False, needs_layout_passes=False),
        scratch_types=[
            pltpu.VMEM_SHARED((npad, accw), jnp.float32),
            pltpu.VMEM((16, _CH), jnp.float32),
            pltpu.VMEM((_CH, accw), jnp.float32),
            pltpu.VMEM((_CH,), jnp.int32),
            pltpu.SemaphoreType.DMA,
        ] + set_types() + set_types(),
    )
    def body(q_ref, kv_ref, eidx_ref, zero_ref, out_ref, acc, escr, msg,
             hold, sems, *bufs):
        sets = [bufs[:5], bufs[5:]]
        cid = lax.axis_index("c")
        sid = lax.axis_index("s")
        wid = cid * _NS + sid
        pltpu.sync_copy(zero_ref, acc.at[pl.ds(sid * npc, npc)])
        for r in range(_CH):
            msg[r, pl.ds(d, 16)] = jnp.zeros((16,), jnp.float32)
        plsc.subcore_barrier()

        base0 = wid * epw
        iota = lax.iota(jnp.int32, 16)

        def fire_idx(i, S):
            (_, _, idxv, semi, _) = S
            b = base0 + i * _CH
            pltpu.async_copy(eidx_ref.at[:, pl.ds(b, _CH)], idxv, semi)

        def wait_idx(i, S):
            (_, _, idxv, semi, _) = S
            b = base0 + i * _CH
            pltpu.make_async_copy(eidx_ref.at[:, pl.ds(b, _CH)], idxv,
                                  semi).wait()

        def fire_gathers(S):
            (qr, kvr, idxv, _, semg) = S
            pltpu.async_copy(q_ref.at[idxv.at[1]], qr, semg)
            pltpu.async_copy(kv_ref.at[idxv.at[0]], kvr, semg)

        def wait_gathers(S):
            (qr, kvr, idxv, _, semg) = S
            pltpu.make_async_copy(q_ref.at[idxv.at[1]], qr, semg).wait()
            pltpu.make_async_copy(kv_ref.at[idxv.at[0]], kvr, semg).wait()

        def wait_scatter():
            pltpu.make_async_copy(msg, acc.at[hold], sems).wait()

        def compute_and_scatter(S):
            (qr, kvr, idxv, _, _) = S
            for cc in range(_CH // 16):
                hold[pl.ds(cc * 16, 16)] = idxv[2, pl.ds(cc * 16, 16)]

            @pl.loop(0, ngrp)
            def _grp(g):
                rows = g * _GRP + iota
                for hh in range(h):
                    s = jnp.zeros((16,), jnp.float32)
                    colr = iota
                    for j in range(dh):
                        col = hh * dh + colr
                        colr = (colr + 1) & (dh - 1)
                        a = plsc.load_gather(qr, [rows, col])
                        b = plsc.load_gather(kvr, [rows, col])
                        s = s + a * b
                    eh = jnp.exp(s)
                    plsc.store_scatter(msg, [rows, jnp.full((16,), d + hh,
                                                            jnp.int32)], eh)
                    plsc.store_scatter(escr, [jnp.full((16,), hh, jnp.int32),
                                              rows], eh)

            @pl.loop(0, ngrp)
            def _grp2(g):
                rows = g * _GRP + iota
                for hh in range(h):
                    eh = plsc.load_gather(escr,
                                          [jnp.full((16,), hh, jnp.int32),
                                           rows])
                    colr = iota
                    for j in range(dh):
                        col = hh * dh + colr
                        colr = (colr + 1) & (dh - 1)
                        vv = plsc.load_gather(kvr, [rows, d + col])
                        plsc.store_scatter(msg, [rows, col], eh * vv)
            pltpu.async_copy(msg, acc.at[hold], sems, add=True)

        fire_idx(0, sets[0])
        fire_idx(1, sets[1])
        wait_idx(0, sets[0])
        fire_gathers(sets[0])

        @pl.loop(0, nchunk // 2)
        def _pair(it):
            for ph in range(2):
                S, Sn = sets[ph], sets[1 - ph]
                i = it * 2 + ph
                wait_gathers(S)

                @pl.when(i >= 1)
                def _():
                    wait_scatter()

                @pl.when(i + 1 < nchunk)
                def _():
                    wait_idx(i + 1, Sn)
                    fire_gathers(Sn)

                @pl.when(i + 2 < nchunk)
                def _():
                    fire_idx(i + 2, S)

                compute_and_scatter(S)

        wait_scatter()
        plsc.subcore_barrier()
        pltpu.sync_copy(acc.at[pl.ds(sid * npc, npc)],
                        out_ref.at[cid, pl.ds(sid * npc, npc)])

    return body(q, kv, eidx, zeros)


def _tc_dense(x, acc2, wo, ga, ba, g1, be1, w1, bb1, w2, bb2, g2, be2):
    n, d = x.shape
    h = 8
    dh = d // h

    def body(x_ref, acc_ref, wo_ref, ga_ref, ba_ref, g1_ref, be1_ref,
             w1_ref, bb1_ref, w2_ref, bb2_ref, g2_ref, be2_ref, out_ref):
        xv = x_ref[...]
        u = acc_ref[0, :n, :d] + acc_ref[1, :n, :d]
        den = acc_ref[0, :n, d:d + h] + acc_ref[1, :n, d:d + h]
        ind = (lax.broadcasted_iota(jnp.int32, (h, d), 1) // dh
               == lax.broadcasted_iota(jnp.int32, (h, d), 0)
               ).astype(jnp.float32)
        denf = jnp.dot(den, ind, preferred_element_type=jnp.float32)
        agg = u / (denf + 1e-16)
        y0 = jnp.dot(agg, wo_ref[...], preferred_element_type=jnp.float32) + xv
        a1 = _bn(y0, ga_ref[...], ba_ref[...])
        x1 = _bn(a1 + xv, g1_ref[...], be1_ref[...])
        hid = jnp.maximum(
            jnp.dot(x1, w1_ref[...], preferred_element_type=jnp.float32)
            + bb1_ref[...], 0.0)
        h2 = jnp.dot(hid, w2_ref[...],
                     preferred_element_type=jnp.float32) + bb2_ref[...]
        out_ref[...] = _bn(h2 + x1, g2_ref[...], be2_ref[...])

    return pl.pallas_call(
        body,
        out_shape=jax.ShapeDtypeStruct((n, d), jnp.float32),
    )(x, acc2, wo, ga, ba, g1, be1, w1, bb1, w2, bb2, g2, be2)


def _tc_final(x, wp, bp):
    n, d = x.shape
    c = wp.shape[1]

    def body(x_ref, wp_ref, bp_ref, out_ref):
        out_ref[...] = (jnp.dot(x_ref[...], wp_ref[...],
                                preferred_element_type=jnp.float32)
                        + bp_ref[...])

    return pl.pallas_call(
        body,
        out_shape=jax.ShapeDtypeStruct((n, c), jnp.float32),
    )(x, wp, bp)


def kernel(x, edge_index, Wq, Wk, Wv, Wo, gamma_attn, beta_attn, gamma1,
           beta1, W1, b1, W2, b2, gamma2, beta2, Wp, bp):
    n, d = x.shape
    nl = Wq.shape[0]
    h = 8
    dh = d // h
    accw = d + 16
    scale = 1.0 / (dh ** 0.5)
    src = edge_index[0]
    dst = edge_index[1]
    e = src.shape[0]
    ep = -(-e // (_NW * 2 * _CH)) * (_NW * 2 * _CH)
    pad = ep - e
    srcp = jnp.concatenate([src, jnp.zeros((pad,), jnp.int32)])
    dstg = jnp.concatenate([dst, jnp.zeros((pad,), jnp.int32)])
    dsts = jnp.concatenate([dst, jnp.full((pad,), n, jnp.int32)])
    eidx = jnp.stack([srcp, dstg, dsts])
    npad = ((n + 127) // 128) * 128
    zeros = jnp.zeros((npad // _NS, accw), jnp.float32)
    for l in range(nl):
        q, kv = _tc_qkv(x, Wq[l], Wk[l], Wv[l], scale)
        acc2 = _sc_edge_pass(q, kv, eidx, zeros, n, d, h, accw)
        x = _tc_dense(x, acc2, Wo[l], gamma_attn[l], beta_attn[l],
                      gamma1[l], beta1[l], W1[l], b1[l], W2[l], b2[l],
                      gamma2[l], beta2[l])
    return _tc_final(x, Wp, bp)


# speedup vs baseline: 2.3968x; 1.0729x over previous
import functools

import jax
import jax.numpy as jnp
from jax import lax
from jax.experimental import pallas as pl
from jax.experimental.pallas import tpu as pltpu
from jax.experimental.pallas import tpu_sc as plsc

_NC = 2
_NS = 16
_NW = _NC * _NS
_CH = 32
_GRP = 16


def _bn(y, g, b):
    m = jnp.mean(y, axis=0)
    v = jnp.mean((y - m) ** 2, axis=0)
    return (y - m) / jnp.sqrt(v + 1e-5) * g + b


def _tc_qkv(x, wq, wk, wv, scale):
    n, d = x.shape

    def body(x_ref, wq_ref, wk_ref, wv_ref, q_ref, kv_ref):
        xv = x_ref[...]
        q_ref[...] = jnp.dot(xv, wq_ref[...],
                             preferred_element_type=jnp.float32) * scale
        kv_ref[:, :d] = jnp.dot(xv, wk_ref[...],
                                preferred_element_type=jnp.float32)
        kv_ref[:, d:] = jnp.dot(xv, wv_ref[...],
                                preferred_element_type=jnp.float32)

    return pl.pallas_call(
        body,
        out_shape=(jax.ShapeDtypeStruct((n, d), jnp.float32),
                   jax.ShapeDtypeStruct((n, 2 * d), jnp.float32)),
    )(x, wq, wk, wv)


def _sc_edge_pass(q, kv, eidx, zeros, n, d, h, accw):
    ep = eidx.shape[1]
    epw = ep // _NW
    nchunk = epw // _CH
    ngrp = _CH // _GRP
    npad = ((n + 127) // 128) * 128
    npc = npad // _NS
    dh = d // h
    mesh = plsc.VectorSubcoreMesh(core_axis_name="c", subcore_axis_name="s")

    def set_types():
        return [
            pltpu.VMEM((_CH, d), jnp.float32),
            pltpu.VMEM((_CH, 2 * d), jnp.float32),
            pltpu.VMEM((3, _CH), jnp.int32),
            pltpu.SemaphoreType.DMA,
            pltpu.SemaphoreType.DMA,
        ]

    @functools.partial(
        pl.kernel,
        out_type=jax.ShapeDtypeStruct((_NC, npad, accw), jnp.float32),
        mesh=mesh,
        compiler_params=pltpu.CompilerParams(
            use_tc_tiling_on_sc=False, needs_layout_passes=False),
        scratch_types=[
            pltpu.VMEM_SHARED((npad, accw), jnp.float32),
            pltpu.VMEM((16, _CH), jnp.float32),
            pltpu.VMEM((_CH, accw), jnp.float32),
            pltpu.VMEM((_CH,), jnp.int32),
            pltpu.SemaphoreType.DMA,
        ] + set_types() + set_types(),
    )
    def body(q_ref, kv_ref, eidx_ref, zero_ref, out_ref, acc, escr, msg,
             hold, sems, *bufs):
        sets = [bufs[:5], bufs[5:]]
        cid = lax.axis_index("c")
        sid = lax.axis_index("s")
        wid = cid * _NS + sid
        pltpu.sync_copy(zero_ref, acc.at[pl.ds(sid * npc, npc)])
        for r in range(_CH):
            msg[r, pl.ds(d, 16)] = jnp.zeros((16,), jnp.float32)
        plsc.subcore_barrier()

        base0 = wid * epw
        iota = lax.iota(jnp.int32, 16)

        def fire_idx(i, S):
            (_, _, idxv, semi, _) = S
            b = base0 + i * _CH
            pltpu.async_copy(eidx_ref.at[:, pl.ds(b, _CH)], idxv, semi)

        def wait_idx(i, S):
            (_, _, idxv, semi, _) = S
            b = base0 + i * _CH
            pltpu.make_async_copy(eidx_ref.at[:, pl.ds(b, _CH)], idxv,
                                  semi).wait()

        def fire_gathers(S):
            (qr, kvr, idxv, _, semg) = S
            pltpu.async_copy(q_ref.at[idxv.at[1]], qr, semg)
            pltpu.async_copy(kv_ref.at[idxv.at[0]], kvr, semg)

        def wait_gathers(S):
            (qr, kvr, idxv, _, semg) = S
            pltpu.make_async_copy(q_ref.at[idxv.at[1]], qr, semg).wait()
            pltpu.make_async_copy(kv_ref.at[idxv.at[0]], kvr, semg).wait()

        def wait_scatter():
            pltpu.make_async_copy(msg, acc.at[hold], sems).wait()

        def compute_and_scatter(S):
            (qr, kvr, idxv, _, _) = S
            for cc in range(_CH // 16):
                hold[pl.ds(cc * 16, 16)] = idxv[2, pl.ds(cc * 16, 16)]

            @pl.loop(0, ngrp)
            def _grp(g):
                rows = g * _GRP + iota
                for hh in range(h):
                    s = jnp.zeros((16,), jnp.float32)
                    colr = iota
                    for j in range(dh):
                        col = hh * dh + colr
                        colr = (colr + 1) & (dh - 1)
                        a = plsc.load_gather(qr, [rows, col])
                        b = plsc.load_gather(kvr, [rows, col])
                        s = s + a * b
                    eh = jnp.exp(s)
                    plsc.store_scatter(msg, [rows, jnp.full((16,), d + hh,
                                                            jnp.int32)], eh)
                    colr2 = iota
                    for j in range(dh):
                        col = hh * dh + colr2
                        colr2 = (colr2 + 1) & (dh - 1)
                        vv = plsc.load_gather(kvr, [rows, d + col])
                        plsc.store_scatter(msg, [rows, col], eh * vv)
            pltpu.async_copy(msg, acc.at[hold], sems, add=True)

        fire_idx(0, sets[0])
        fire_idx(1, sets[1])
        wait_idx(0, sets[0])
        fire_gathers(sets[0])

        @pl.loop(0, nchunk // 2)
        def _pair(it):
            for ph in range(2):
                S, Sn = sets[ph], sets[1 - ph]
                i = it * 2 + ph
                wait_gathers(S)

                @pl.when(i >= 1)
                def _():
                    wait_scatter()

                @pl.when(i + 1 < nchunk)
                def _():
                    wait_idx(i + 1, Sn)
                    fire_gathers(Sn)

                @pl.when(i + 2 < nchunk)
                def _():
                    fire_idx(i + 2, S)

                compute_and_scatter(S)

        wait_scatter()
        plsc.subcore_barrier()
        pltpu.sync_copy(acc.at[pl.ds(sid * npc, npc)],
                        out_ref.at[cid, pl.ds(sid * npc, npc)])

    return body(q, kv, eidx, zeros)


def _tc_dense(x, acc2, wo, ga, ba, g1, be1, w1, bb1, w2, bb2, g2, be2):
    n, d = x.shape
    h = 8
    dh = d // h

    def body(x_ref, acc_ref, wo_ref, ga_ref, ba_ref, g1_ref, be1_ref,
             w1_ref, bb1_ref, w2_ref, bb2_ref, g2_ref, be2_ref, out_ref):
        xv = x_ref[...]
        u = acc_ref[0, :n, :d] + acc_ref[1, :n, :d]
        den = acc_ref[0, :n, d:d + h] + acc_ref[1, :n, d:d + h]
        ind = (lax.broadcasted_iota(jnp.int32, (h, d), 1) // dh
               == lax.broadcasted_iota(jnp.int32, (h, d), 0)
               ).astype(jnp.float32)
        denf = jnp.dot(den, ind, preferred_element_type=jnp.float32)
        agg = u / (denf + 1e-16)
        y0 = jnp.dot(agg, wo_ref[...], preferred_element_type=jnp.float32) + xv
        a1 = _bn(y0, ga_ref[...], ba_ref[...])
        x1 = _bn(a1 + xv, g1_ref[...], be1_ref[...])
        hid = jnp.maximum(
            jnp.dot(x1, w1_ref[...], preferred_element_type=jnp.float32)
            + bb1_ref[...], 0.0)
        h2 = jnp.dot(hid, w2_ref[...],
                     preferred_element_type=jnp.float32) + bb2_ref[...]
        out_ref[...] = _bn(h2 + x1, g2_ref[...], be2_ref[...])

    return pl.pallas_call(
        body,
        out_shape=jax.ShapeDtypeStruct((n, d), jnp.float32),
    )(x, acc2, wo, ga, ba, g1, be1, w1, bb1, w2, bb2, g2, be2)


def _tc_final(x, wp, bp):
    n, d = x.shape
    c = wp.shape[1]

    def body(x_ref, wp_ref, bp_ref, out_ref):
        out_ref[...] = (jnp.dot(x_ref[...], wp_ref[...],
                                preferred_element_type=jnp.float32)
                        + bp_ref[...])

    return pl.pallas_call(
        body,
        out_shape=jax.ShapeDtypeStruct((n, c), jnp.float32),
    )(x, wp, bp)


def kernel(x, edge_index, Wq, Wk, Wv, Wo, gamma_attn, beta_attn, gamma1,
           beta1, W1, b1, W2, b2, gamma2, beta2, Wp, bp):
    n, d = x.shape
    nl = Wq.shape[0]
    h = 8
    dh = d // h
    accw = d + 16
    scale = 1.0 / (dh ** 0.5)
    src = edge_index[0]
    dst = edge_index[1]
    e = src.shape[0]
    ep = -(-e // (_NW * 2 * _CH)) * (_NW * 2 * _CH)
    pad = ep - e
    srcp = jnp.concatenate([src, jnp.zeros((pad,), jnp.int32)])
    dstg = jnp.concatenate([dst, jnp.zeros((pad,), jnp.int32)])
    dsts = jnp.concatenate([dst, jnp.full((pad,), n, jnp.int32)])
    eidx = jnp.stack([srcp, dstg, dsts])
    npad = ((n + 127) // 128) * 128
    zeros = jnp.zeros((npad // _NS, accw), jnp.float32)
    for l in range(nl):
        q, kv = _tc_qkv(x, Wq[l], Wk[l], Wv[l], scale)
        acc2 = _sc_edge_pass(q, kv, eidx, zeros, n, d, h, accw)
        x = _tc_dense(x, acc2, Wo[l], gamma_attn[l], beta_attn[l],
                      gamma1[l], beta1[l], W1[l], b1[l], W2[l], b2[l],
                      gamma2[l], beta2[l])
    return _tc_final(x, Wp, bp)


# speedup vs baseline: 2.3989x; 1.0009x over previous
import functools

import jax
import jax.numpy as jnp
from jax import lax
from jax.experimental import pallas as pl
from jax.experimental.pallas import tpu as pltpu
from jax.experimental.pallas import tpu_sc as plsc

_NC = 2
_NS = 16
_NW = _NC * _NS
_CH = 32
_GRP = 16


def _bn(y, g, b):
    m = jnp.mean(y, axis=0)
    v = jnp.mean((y - m) ** 2, axis=0)
    return (y - m) / jnp.sqrt(v + 1e-5) * g + b


def _tc_qkv(x, wq, wk, wv, scale):
    n, d = x.shape

    def body(x_ref, wq_ref, wk_ref, wv_ref, q_ref, kv_ref):
        xv = x_ref[...]
        q_ref[...] = jnp.dot(xv, wq_ref[...],
                             preferred_element_type=jnp.float32) * scale
        kv_ref[:, :d] = jnp.dot(xv, wk_ref[...],
                                preferred_element_type=jnp.float32)
        kv_ref[:, d:] = jnp.dot(xv, wv_ref[...],
                                preferred_element_type=jnp.float32)

    return pl.pallas_call(
        body,
        out_shape=(jax.ShapeDtypeStruct((n, d), jnp.float32),
                   jax.ShapeDtypeStruct((n, 2 * d), jnp.float32)),
    )(x, wq, wk, wv)


def _sc_edge_pass(q, kv, eidx, zeros, n, d, h, accw):
    ep = eidx.shape[1]
    epw = ep // _NW
    nchunk = epw // _CH
    ngrp = _CH // _GRP
    npad = ((n + 127) // 128) * 128
    npc = npad // _NS
    dh = d // h
    mesh = plsc.VectorSubcoreMesh(core_axis_name="c", subcore_axis_name="s")

    def set_types():
        return [
            pltpu.VMEM((_CH, d), jnp.float32),
            pltpu.VMEM((_CH, 2 * d), jnp.float32),
            pltpu.VMEM((3, _CH), jnp.int32),
            pltpu.SemaphoreType.DMA,
            pltpu.SemaphoreType.DMA,
        ]

    @functools.partial(
        pl.kernel,
        out_type=jax.ShapeDtypeStruct((_NC, npad, accw), jnp.float32),
        mesh=mesh,
        compiler_params=pltpu.CompilerParams(
            use_tc_tiling_on_sc=False, needs_layout_passes=False),
        scratch_types=[
            pltpu.VMEM_SHARED((npad, accw), jnp.float32),
            pltpu.VMEM((_CH, accw), jnp.float32),
            pltpu.VMEM((_CH,), jnp.int32),
            pltpu.SemaphoreType.DMA,
        ] + set_types() + set_types(),
    )
    def body(q_ref, kv_ref, eidx_ref, zero_ref, out_ref, acc, msg,
             hold, sems, *bufs):
        sets = [bufs[:5], bufs[5:]]
        cid = lax.axis_index("c")
        sid = lax.axis_index("s")
        wid = cid * _NS + sid
        pltpu.sync_copy(zero_ref, acc.at[pl.ds(sid * npc, npc)])
        for r in range(_CH):
            msg[r, pl.ds(d, 16)] = jnp.zeros((16,), jnp.float32)
        plsc.subcore_barrier()

        base0 = wid * epw
        iota = lax.iota(jnp.int32, 16)

        def fire_idx(i, S):
            (_, _, idxv, semi, _) = S
            b = base0 + i * _CH
            pltpu.async_copy(eidx_ref.at[:, pl.ds(b, _CH)], idxv, semi)

        def wait_idx(i, S):
            (_, _, idxv, semi, _) = S
            b = base0 + i * _CH
            pltpu.make_async_copy(eidx_ref.at[:, pl.ds(b, _CH)], idxv,
                                  semi).wait()

        def fire_gathers(S):
            (qr, kvr, idxv, _, semg) = S
            pltpu.async_copy(q_ref.at[idxv.at[1]], qr, semg)
            pltpu.async_copy(kv_ref.at[idxv.at[0]], kvr, semg)

        def wait_gathers(S):
            (qr, kvr, idxv, _, semg) = S
            pltpu.make_async_copy(q_ref.at[idxv.at[1]], qr, semg).wait()
            pltpu.make_async_copy(kv_ref.at[idxv.at[0]], kvr, semg).wait()

        def wait_scatter():
            pltpu.make_async_copy(msg, acc.at[hold], sems).wait()

        def compute_and_scatter(S):
            (qr, kvr, idxv, _, _) = S
            for cc in range(_CH // 16):
                hold[pl.ds(cc * 16, 16)] = idxv[2, pl.ds(cc * 16, 16)]

            @pl.loop(0, ngrp)
            def _grp(g):
                rows = g * _GRP + iota
                for hh in range(h):
                    s = jnp.zeros((16,), jnp.float32)
                    colr = iota
                    for j in range(dh):
                        col = hh * dh + colr
                        colr = (colr + 1) & (dh - 1)
                        a = plsc.load_gather(qr, [rows, col])
                        b = plsc.load_gather(kvr, [rows, col])
                        s = s + a * b
                    eh = jnp.exp(s)
                    plsc.store_scatter(msg, [rows, jnp.full((16,), d + hh,
                                                            jnp.int32)], eh)
                    colr2 = iota
                    for j in range(dh):
                        col = hh * dh + colr2
                        colr2 = (colr2 + 1) & (dh - 1)
                        vv = plsc.load_gather(kvr, [rows, d + col])
                        plsc.store_scatter(msg, [rows, col], eh * vv)
            pltpu.async_copy(msg, acc.at[hold], sems, add=True)

        fire_idx(0, sets[0])
        fire_idx(1, sets[1])
        wait_idx(0, sets[0])
        fire_gathers(sets[0])

        @pl.loop(0, nchunk // 2)
        def _pair(it):
            for ph in range(2):
                S, Sn = sets[ph], sets[1 - ph]
                i = it * 2 + ph
                wait_gathers(S)

                @pl.when(i >= 1)
                def _():
                    wait_scatter()

                @pl.when(i + 1 < nchunk)
                def _():
                    wait_idx(i + 1, Sn)
                    fire_gathers(Sn)

                @pl.when(i + 2 < nchunk)
                def _():
                    fire_idx(i + 2, S)

                compute_and_scatter(S)

        wait_scatter()
        plsc.subcore_barrier()
        pltpu.sync_copy(acc.at[pl.ds(sid * npc, npc)],
                        out_ref.at[cid, pl.ds(sid * npc, npc)])

    return body(q, kv, eidx, zeros)


def _tc_dense(x, acc2, wo, ga, ba, g1, be1, w1, bb1, w2, bb2, g2, be2):
    n, d = x.shape
    h = 8
    dh = d // h

    def body(x_ref, acc_ref, wo_ref, ga_ref, ba_ref, g1_ref, be1_ref,
             w1_ref, bb1_ref, w2_ref, bb2_ref, g2_ref, be2_ref, out_ref):
        xv = x_ref[...]
        u = acc_ref[0, :n, :d] + acc_ref[1, :n, :d]
        den = acc_ref[0, :n, d:d + h] + acc_ref[1, :n, d:d + h]
        ind = (lax.broadcasted_iota(jnp.int32, (h, d), 1) // dh
               == lax.broadcasted_iota(jnp.int32, (h, d), 0)
               ).astype(jnp.float32)
        denf = jnp.dot(den, ind, preferred_element_type=jnp.float32)
        agg = u / (denf + 1e-16)
        y0 = jnp.dot(agg, wo_ref[...], preferred_element_type=jnp.float32) + xv
        a1 = _bn(y0, ga_ref[...], ba_ref[...])
        x1 = _bn(a1 + xv, g1_ref[...], be1_ref[...])
        hid = jnp.maximum(
            jnp.dot(x1, w1_ref[...], preferred_element_type=jnp.float32)
            + bb1_ref[...], 0.0)
        h2 = jnp.dot(hid, w2_ref[...],
                     preferred_element_type=jnp.float32) + bb2_ref[...]
        out_ref[...] = _bn(h2 + x1, g2_ref[...], be2_ref[...])

    return pl.pallas_call(
        body,
        out_shape=jax.ShapeDtypeStruct((n, d), jnp.float32),
    )(x, acc2, wo, ga, ba, g1, be1, w1, bb1, w2, bb2, g2, be2)


def _tc_final(x, wp, bp):
    n, d = x.shape
    c = wp.shape[1]

    def body(x_ref, wp_ref, bp_ref, out_ref):
        out_ref[...] = (jnp.dot(x_ref[...], wp_ref[...],
                                preferred_element_type=jnp.float32)
                        + bp_ref[...])

    return pl.pallas_call(
        body,
        out_shape=jax.ShapeDtypeStruct((n, c), jnp.float32),
    )(x, wp, bp)


def kernel(x, edge_index, Wq, Wk, Wv, Wo, gamma_attn, beta_attn, gamma1,
           beta1, W1, b1, W2, b2, gamma2, beta2, Wp, bp):
    n, d = x.shape
    nl = Wq.shape[0]
    h = 8
    dh = d // h
    accw = d + 16
    scale = 1.0 / (dh ** 0.5)
    src = edge_index[0]
    dst = edge_index[1]
    e = src.shape[0]
    ep = -(-e // (_NW * 2 * _CH)) * (_NW * 2 * _CH)
    pad = ep - e
    srcp = jnp.concatenate([src, jnp.zeros((pad,), jnp.int32)])
    dstg = jnp.concatenate([dst, jnp.zeros((pad,), jnp.int32)])
    dsts = jnp.concatenate([dst, jnp.full((pad,), n, jnp.int32)])
    eidx = jnp.stack([srcp, dstg, dsts])
    npad = ((n + 127) // 128) * 128
    zeros = jnp.zeros((npad // _NS, accw), jnp.float32)
    for l in range(nl):
        q, kv = _tc_qkv(x, Wq[l], Wk[l], Wv[l], scale)
        acc2 = _sc_edge_pass(q, kv, eidx, zeros, n, d, h, accw)
        x = _tc_dense(x, acc2, Wo[l], gamma_attn[l], beta_attn[l],
                      gamma1[l], beta1[l], W1[l], b1[l], W2[l], b2[l],
                      gamma2[l], beta2[l])
    return _tc_final(x, Wp, bp)
